# baseline (speedup 1.0000x reference)
import functools

import jax
import jax.numpy as jnp
from jax import lax
from jax.experimental import pallas as pl
from jax.experimental.pallas import tpu as pltpu
from jax.experimental.pallas import tpu_sc as plsc

R = 128
C = 32768
L = 16
NC = 2
NS = 16
NW = NC * NS
ROWS_PER_W = R // NW
U = 8
STEPS = C // L

_mesh = plsc.VectorSubcoreMesh(core_axis_name="c", subcore_axis_name="s")


def _shuffle(x, idx):
    return lax.gather(
        x, idx[:, None],
        lax.GatherDimensionNumbers(
            offset_dims=(), collapsed_slice_dims=(0,), start_index_map=(0,)),
        slice_sizes=(1,),
        mode=lax.GatherScatterMode.PROMISE_IN_BOUNDS)


@functools.partial(
    pl.kernel,
    out_type=jax.ShapeDtypeStruct((NW, L), jnp.int32),
    mesh=_mesh,
    scratch_types=[
        pltpu.VMEM((C,), jnp.float32),
        pltpu.VMEM((C,), jnp.float32),
        pltpu.VMEM((L,), jnp.int32),
        pltpu.SemaphoreType.DMA,
        pltpu.SemaphoreType.DMA,
    ],
    compiler_params=pltpu.CompilerParams(needs_layout_passes=False),
)
def _row_argmax(data_hbm, idx_hbm, in0, in1, idx_v, sem0, sem1):
    wid = lax.axis_index("s") * NC + lax.axis_index("c")
    lanes = lax.iota(jnp.int32, L)
    bufs = (in0, in1)
    sems = (sem0, sem1)
    base_row = wid * ROWS_PER_W

    cps = [pltpu.async_copy(data_hbm.at[base_row], in0, sem0), None]
    acc = jnp.zeros((L,), jnp.int32)
    for r in range(ROWS_PER_W):
        cps[r % 2].wait()
        if r + 1 < ROWS_PER_W:
            cps[(r + 1) % 2] = pltpu.async_copy(
                data_hbm.at[base_row + r + 1], bufs[(r + 1) % 2],
                sems[(r + 1) % 2])
        buf = bufs[r % 2]

        def step(t, carry, buf=buf):
            bv, bi = carry
            base = t * (U * L)
            for k in range(U):
                v = buf[pl.ds(base + k * L, L)]
                idx = (base + k * L) + lanes
                upd = v > bv
                bv = jnp.where(upd, v, bv)
                bi = jnp.where(upd, idx, bi)
            return bv, bi

        init = (jnp.full((L,), -jnp.inf, jnp.float32),
                jnp.zeros((L,), jnp.int32))
        bv, bi = lax.fori_loop(0, STEPS // U, step, init)

        for k in (8, 4, 2, 1):
            pv = _shuffle(bv, lanes ^ k)
            pi = _shuffle(bi, lanes ^ k)
            take = (pv > bv) | ((pv == bv) & (pi < bi))
            bv = jnp.where(take, pv, bv)
            bi = jnp.where(take, pi, bi)

        acc = jnp.where(lanes == r, bi, acc)

    idx_v[...] = acc
    pltpu.sync_copy(idx_v, idx_hbm.at[wid])


_ROWS_PER_BLK = 8


def _onehot_body(idx_ref, out_ref):
    i = pl.program_id(0)
    iota = lax.broadcasted_iota(jnp.int32, (1, C), 1)
    for k in range(_ROWS_PER_BLK):
        s = idx_ref[i * _ROWS_PER_BLK + k]
        out_ref[pl.ds(k, 1), :] = (iota == s).astype(jnp.float32)


_onehot = pl.pallas_call(
    _onehot_body,
    grid_spec=pltpu.PrefetchScalarGridSpec(
        num_scalar_prefetch=1,
        grid=(R // _ROWS_PER_BLK,),
        out_specs=pl.BlockSpec((_ROWS_PER_BLK, C), lambda i, idx_ref: (i, 0)),
    ),
    out_shape=jax.ShapeDtypeStruct((R, C), jnp.float32),
)


def kernel(data):
    idx2d = _row_argmax(data)
    idx = idx2d[:, :ROWS_PER_W].reshape(R)
    return _onehot(idx)


# speedup vs baseline: 1.2487x; 1.2487x over previous
import functools

import jax
import jax.numpy as jnp
from jax import lax
from jax.experimental import pallas as pl
from jax.experimental.pallas import tpu as pltpu
from jax.experimental.pallas import tpu_sc as plsc

R = 128
C = 32768
L = 16
NC = 2
NS = 16
NW = NC * NS
ROWS_PER_W = R // NW
U = 8
STEPS = C // L

_mesh = plsc.VectorSubcoreMesh(core_axis_name="c", subcore_axis_name="s")


def _shuffle(x, idx):
    return lax.gather(
        x, idx[:, None],
        lax.GatherDimensionNumbers(
            offset_dims=(), collapsed_slice_dims=(0,), start_index_map=(0,)),
        slice_sizes=(1,),
        mode=lax.GatherScatterMode.PROMISE_IN_BOUNDS)


@functools.partial(
    pl.kernel,
    out_type=jax.ShapeDtypeStruct((NW, L), jnp.int32),
    mesh=_mesh,
    scratch_types=[
        pltpu.VMEM((C,), jnp.float32),
        pltpu.VMEM((C,), jnp.float32),
        pltpu.VMEM((L,), jnp.int32),
        pltpu.SemaphoreType.DMA,
        pltpu.SemaphoreType.DMA,
    ],
    compiler_params=pltpu.CompilerParams(needs_layout_passes=False),
)
def _row_argmax(data_hbm, idx_hbm, in0, in1, idx_v, sem0, sem1):
    wid = lax.axis_index("s") * NC + lax.axis_index("c")
    lanes = lax.iota(jnp.int32, L)
    bufs = (in0, in1)
    sems = (sem0, sem1)
    base_row = wid * ROWS_PER_W

    cps = [pltpu.async_copy(data_hbm.at[base_row], in0, sem0), None]
    acc = jnp.zeros((L,), jnp.int32)
    for r in range(ROWS_PER_W):
        cps[r % 2].wait()
        if r + 1 < ROWS_PER_W:
            cps[(r + 1) % 2] = pltpu.async_copy(
                data_hbm.at[base_row + r + 1], bufs[(r + 1) % 2],
                sems[(r + 1) % 2])
        buf = bufs[r % 2]

        def step(t, carry, buf=buf):
            bv, bi = carry
            base = t * (U * L)
            for k in range(U):
                v = buf[pl.ds(base + k * L, L)]
                idx = (base + k * L) + lanes
                upd = v > bv
                bv = jnp.where(upd, v, bv)
                bi = jnp.where(upd, idx, bi)
            return bv, bi

        init = (jnp.full((L,), -jnp.inf, jnp.float32),
                jnp.zeros((L,), jnp.int32))
        bv, bi = lax.fori_loop(0, STEPS // U, step, init)

        for k in (8, 4, 2, 1):
            pv = _shuffle(bv, lanes ^ k)
            pi = _shuffle(bi, lanes ^ k)
            take = (pv > bv) | ((pv == bv) & (pi < bi))
            bv = jnp.where(take, pv, bv)
            bi = jnp.where(take, pi, bi)

        acc = jnp.where(lanes == r, bi, acc)

    idx_v[...] = acc
    pltpu.sync_copy(idx_v, idx_hbm.at[wid])


_ROWS_PER_BLK = 16


def _onehot_body(idx_ref, out_ref):
    idxv = idx_ref[...]
    col = lax.broadcasted_iota(jnp.int32, (_ROWS_PER_BLK, C), 1)
    out_ref[...] = (col == idxv).astype(jnp.float32)


_onehot = pl.pallas_call(
    _onehot_body,
    grid=(R // _ROWS_PER_BLK,),
    in_specs=[pl.BlockSpec((_ROWS_PER_BLK, 1), lambda i: (i, 0))],
    out_specs=pl.BlockSpec((_ROWS_PER_BLK, C), lambda i: (i, 0)),
    out_shape=jax.ShapeDtypeStruct((R, C), jnp.float32),
)


def kernel(data):
    idx2d = _row_argmax(data)
    idx = idx2d[:, :ROWS_PER_W].reshape(R, 1)
    return _onehot(idx)
